# baseline (speedup 1.0000x reference)
import functools

import jax
import jax.numpy as jnp
from jax import lax
from jax.experimental import pallas as pl
from jax.experimental.pallas import tpu as pltpu
from jax.experimental.pallas import tpu_sc as plsc

_N = 10000
_E = 320000
_HID = 128
_HALF = 64
_OUT = 40
_TILES = 16
_B = 125
_EPT = _E // _TILES
_NB = _EPT // _B
_BW = 80
_EPS_GEN = 1e-7
_LANES = 16


def _sc_edge_softmax(hn2, idx4):
    mesh = plsc.VectorSubcoreMesh(core_axis_name="c", subcore_axis_name="s")
    assert _NB % 6 == 4 and _NB % 2 == 0
    _STEADY = (_NB - 4) // 6

    @functools.partial(
        pl.kernel,
        out_type=jax.ShapeDtypeStruct((2 * _N, _HID), jnp.float32),
        mesh=mesh,
        compiler_params=pltpu.CompilerParams(use_tc_tiling_on_sc=False),
        scratch_types=[
            pltpu.VMEM((6, 2, _B), jnp.int32),
            pltpu.VMEM((2, _B, _HALF), jnp.float32),
            pltpu.VMEM((2, _B, _HID), jnp.float32),
            pltpu.VMEM_SHARED((_N, _HID), jnp.float32),
        ] + [pltpu.SemaphoreType.DMA] * 10,
    )
    def k(hn2_h, idx4_h, out_h, ring, grows, obuf, acc,
          is0, is1, is2, is3, is4, is5, gs0, gs1, ss0, ss1):
        cid = lax.axis_index("c")
        tid = lax.axis_index("s")
        isems = (is0, is1, is2, is3, is4, is5)
        gsems = (gs0, gs1)
        ssems = (ss0, ss1)

        def _zrow(i, carry):
            for q in range(_HID // _LANES):
                obuf[0, i, pl.ds(q * _LANES, _LANES)] = jnp.zeros((_LANES,), jnp.float32)
            return carry

        lax.fori_loop(0, _BW, _zrow, 0)
        nchunks = _N // _BW
        for q in range(-(-nchunks // _TILES)):
            c = q * _TILES + tid

            @pl.when(c < nchunks)
            def _():
                pltpu.sync_copy(obuf.at[0, pl.ds(0, _BW)], acc.at[pl.ds(c * _BW, _BW)])

        plsc.subcore_barrier()

        def _ring_load(j, slot):
            pltpu.async_copy(idx4_h.at[cid, tid, j], ring.at[slot], isems[slot])

        def _ring_wait(j, slot):
            pltpu.make_async_copy(idx4_h.at[cid, tid, j], ring.at[slot], isems[slot]).wait()

        def _gather(slot, b):
            pltpu.async_copy(hn2_h.at[ring.at[slot, 0]], grows.at[b], gsems[b])

        def _gather_wait(slot, b):
            pltpu.make_async_copy(hn2_h.at[ring.at[slot, 0]], grows.at[b], gsems[b]).wait()

        def _scatter(slot, b):
            pltpu.async_copy(obuf.at[b], acc.at[ring.at[slot, 1]], ssems[b], add=True)

        def _scatter_wait(slot, b):
            pltpu.make_async_copy(obuf.at[b], acc.at[ring.at[slot, 1]], ssems[b]).wait()

        def _slot(j, u, *, guard_first, load_ahead, gather_ahead):
            b = u % 2
            _gather_wait(u, b)
            if guard_first:
                @pl.when(j >= 2)
                def _():
                    _scatter_wait((u - 2) % 6, b)
            else:
                _scatter_wait((u - 2) % 6, b)
            if load_ahead:
                _ring_load(j + 4, (u + 4) % 6)

            @plsc.parallel_loop(0, _B, unroll=2)
            def _row(i):
                for q in range(_HALF // _LANES):
                    g = grows[b, i, pl.ds(q * _LANES, _LANES)]
                    m = g + _EPS_GEN
                    p = jnp.exp(m)
                    obuf[b, i, pl.ds(q * _LANES, _LANES)] = m * p
                    obuf[b, i, pl.ds(_HALF + q * _LANES, _LANES)] = p

            if gather_ahead:
                _ring_wait(j + 2, (u + 2) % 6)
                _gather((u + 2) % 6, b)
            _scatter(u, b)

        for t in range(4):
            _ring_load(t, t)
        for t in range(2):
            _ring_wait(t, t)
            _gather(t, t)

        def _six(kk, carry):
            for u in range(6):
                _slot(6 * kk + u, u, guard_first=(u < 2), load_ahead=True,
                      gather_ahead=True)
            return carry

        lax.fori_loop(0, _STEADY, _six, 0)
        for j in range(6 * _STEADY, _NB):
            _slot(j, j % 6, guard_first=False, load_ahead=False,
                  gather_ahead=(j + 2 < _NB))
        _scatter_wait((_NB - 2) % 6, 0)
        _scatter_wait((_NB - 1) % 6, 1)
        plsc.subcore_barrier()

        for q in range(-(-nchunks // _TILES)):
            c = q * _TILES + tid

            @pl.when(c < nchunks)
            def _():
                pltpu.sync_copy(acc.at[pl.ds(c * _BW, _BW)], obuf.at[0, pl.ds(0, _BW)])
                pltpu.sync_copy(obuf.at[0, pl.ds(0, _BW)], out_h.at[pl.ds(cid * _N + c * _BW, _BW)])

    return k(hn2, idx4)


def _norm_relu(h, g, bb):
    mean = jnp.mean(h, axis=0, keepdims=True)
    var = jnp.mean((h - mean) ** 2, axis=0, keepdims=True)
    hn = (h - mean) * lax.rsqrt(var + 1e-5) * g + bb
    return jnp.maximum(hn, 0.0)


def _enc_body(x_ref, w_ref, b_ref, g_ref, bb_ref, h_ref, hn2_ref):
    h = jnp.dot(x_ref[...], w_ref[...], preferred_element_type=jnp.float32) + b_ref[...]
    h_ref[...] = h
    hn = _norm_relu(h, g_ref[...], bb_ref[...])
    hn2_ref[pl.ds(0, _N), :] = hn[:, :_HALF]
    hn2_ref[pl.ds(_N, _N), :] = hn[:, _HALF:]


def _agg_feats(s_ref, hn2_ref):
    s = s_ref[...]
    hn2 = hn2_ref[...]
    f0 = hn2[:_N] + s[:_N, :_HALF] / (s[:_N, _HALF:] + 1e-16)
    f1 = hn2[_N:] + s[_N:, :_HALF] / (s[_N:, _HALF:] + 1e-16)
    return f0, f1


def _mid_body(s_ref, hn2_ref, h_ref, w_ref, b_ref, g_ref, bb_ref, h_out, hn2_out):
    f0, f1 = _agg_feats(s_ref, hn2_ref)
    conv = (
        jnp.dot(f0, w_ref[: _HALF, :], preferred_element_type=jnp.float32)
        + jnp.dot(f1, w_ref[_HALF:, :], preferred_element_type=jnp.float32)
        + b_ref[...]
    )
    h_new = h_ref[...] + conv
    h_out[...] = h_new
    hn_new = _norm_relu(h_new, g_ref[...], bb_ref[...])
    hn2_out[pl.ds(0, _N), :] = hn_new[:, :_HALF]
    hn2_out[pl.ds(_N, _N), :] = hn_new[:, _HALF:]


def _last_body(s_ref, hn2_ref, h_ref, w_ref, b_ref, wo_ref, bo_ref, out_ref):
    f0, f1 = _agg_feats(s_ref, hn2_ref)
    conv = (
        jnp.dot(f0, w_ref[: _HALF, :], preferred_element_type=jnp.float32)
        + jnp.dot(f1, w_ref[_HALF:, :], preferred_element_type=jnp.float32)
        + b_ref[...]
    )
    h_new = h_ref[...] + conv
    logits = jnp.dot(h_new, wo_ref[...], preferred_element_type=jnp.float32) + bo_ref[...]
    mx = jnp.max(logits, axis=-1, keepdims=True)
    sh = logits - mx
    lse = jnp.log(jnp.sum(jnp.exp(sh), axis=-1, keepdims=True))
    out_ref[...] = sh - lse


def kernel(node_feats, edge_index, W_enc, b_enc, gamma, beta_bn, W_conv, b_conv, W_out, b_out):
    src = edge_index[0]
    dst = edge_index[1]
    src3 = jnp.stack([src, src + _N]).reshape(2, _TILES, _NB, _B)
    dst3 = jnp.broadcast_to(dst.reshape(1, _TILES, _NB, _B), (2, _TILES, _NB, _B))
    idx4 = jnp.stack([src3, dst3], axis=3)
    b_enc2 = b_enc.reshape(1, _HID)
    gamma2 = gamma.reshape(1, _HID)
    beta2 = beta_bn.reshape(1, _HID)

    h, hn2 = pl.pallas_call(
        _enc_body,
        out_shape=(
            jax.ShapeDtypeStruct((_N, _HID), jnp.float32),
            jax.ShapeDtypeStruct((2 * _N, _HALF), jnp.float32),
        ),
    )(node_feats, W_enc, b_enc2, gamma2, beta2)

    for i in range(3):
        s = _sc_edge_softmax(hn2, idx4)
        h, hn2 = pl.pallas_call(
            _mid_body,
            out_shape=(
                jax.ShapeDtypeStruct((_N, _HID), jnp.float32),
                jax.ShapeDtypeStruct((2 * _N, _HALF), jnp.float32),
            ),
        )(s, hn2, h, W_conv[i], b_conv[i].reshape(1, _HID), gamma2, beta2)

    s = _sc_edge_softmax(hn2, idx4)
    return pl.pallas_call(
        _last_body,
        out_shape=jax.ShapeDtypeStruct((_N, _OUT), jnp.float32),
    )(s, hn2, h, W_conv[3], b_conv[3].reshape(1, _HID), W_out, b_out.reshape(1, _OUT))


# speedup vs baseline: 1.0502x; 1.0502x over previous
import functools

import jax
import jax.numpy as jnp
from jax import lax
from jax.experimental import pallas as pl
from jax.experimental.pallas import tpu as pltpu
from jax.experimental.pallas import tpu_sc as plsc

_N = 10000
_E = 320000
_HID = 128
_HALF = 64
_OUT = 40
_TILES = 16
_B = 80
_EPT = _E // _TILES
_NB = _EPT // _B
_BW = 80
_EPS_GEN = 1e-7
_LANES = 16


def _sc_edge_softmax(hn2, idx4):
    mesh = plsc.VectorSubcoreMesh(core_axis_name="c", subcore_axis_name="s")
    assert _NB % 6 == 4 and _NB % 2 == 0
    _STEADY = (_NB - 4) // 6

    @functools.partial(
        pl.kernel,
        out_type=jax.ShapeDtypeStruct((2 * _N, _HID), jnp.float32),
        mesh=mesh,
        compiler_params=pltpu.CompilerParams(use_tc_tiling_on_sc=False),
        scratch_types=[
            pltpu.VMEM((6, 2, _B), jnp.int32),
            pltpu.VMEM((3, _B, _HALF), jnp.float32),
            pltpu.VMEM((2, _B, _HID), jnp.float32),
            pltpu.VMEM_SHARED((_N, _HID), jnp.float32),
        ] + [pltpu.SemaphoreType.DMA] * 11,
    )
    def k(hn2_h, idx4_h, out_h, ring, grows, obuf, acc,
          is0, is1, is2, is3, is4, is5, gs0, gs1, gs2, ss0, ss1):
        cid = lax.axis_index("c")
        tid = lax.axis_index("s")
        isems = (is0, is1, is2, is3, is4, is5)
        gsems = (gs0, gs1, gs2)
        ssems = (ss0, ss1)

        def _zrow(i, carry):
            for q in range(_HID // _LANES):
                obuf[0, i, pl.ds(q * _LANES, _LANES)] = jnp.zeros((_LANES,), jnp.float32)
            return carry

        lax.fori_loop(0, _BW, _zrow, 0)
        nchunks = _N // _BW
        for q in range(-(-nchunks // _TILES)):
            c = q * _TILES + tid

            @pl.when(c < nchunks)
            def _():
                pltpu.sync_copy(obuf.at[0, pl.ds(0, _BW)], acc.at[pl.ds(c * _BW, _BW)])

        plsc.subcore_barrier()

        def _ring_load(j, slot):
            pltpu.async_copy(idx4_h.at[cid, tid, j], ring.at[slot], isems[slot])

        def _ring_wait(j, slot):
            pltpu.make_async_copy(idx4_h.at[cid, tid, j], ring.at[slot], isems[slot]).wait()

        def _gather(slot, gb):
            pltpu.async_copy(hn2_h.at[ring.at[slot, 0]], grows.at[gb], gsems[gb])

        def _gather_wait(slot, gb):
            pltpu.make_async_copy(hn2_h.at[ring.at[slot, 0]], grows.at[gb], gsems[gb]).wait()

        def _scatter(slot, b):
            pltpu.async_copy(obuf.at[b], acc.at[ring.at[slot, 1]], ssems[b], add=True)

        def _scatter_wait(slot, b):
            pltpu.make_async_copy(obuf.at[b], acc.at[ring.at[slot, 1]], ssems[b]).wait()

        def _slot(j, u, *, guard_first, load_ahead, gather_ahead):
            gb = u % 3
            b = u % 2
            _gather_wait(u, gb)
            if guard_first:
                @pl.when(j >= 2)
                def _():
                    _scatter_wait((u - 2) % 6, b)
            else:
                _scatter_wait((u - 2) % 6, b)
            if load_ahead:
                _ring_load(j + 4, (u + 4) % 6)
            if gather_ahead:
                _ring_wait(j + 2, (u + 2) % 6)
                _gather((u + 2) % 6, (u + 2) % 3)

            @plsc.parallel_loop(0, _B, unroll=2)
            def _row(i):
                g = grows
                for q in range(_HALF // _LANES):
                    v = g[gb, i, pl.ds(q * _LANES, _LANES)]
                    m = v + _EPS_GEN
                    p = jnp.exp(m)
                    obuf[b, i, pl.ds(q * _LANES, _LANES)] = m * p
                    obuf[b, i, pl.ds(_HALF + q * _LANES, _LANES)] = p

            _scatter(u, b)

        for t in range(4):
            _ring_load(t, t)
        for t in range(2):
            _ring_wait(t, t)
            _gather(t, t)

        def _six(kk, carry):
            for u in range(6):
                _slot(6 * kk + u, u, guard_first=(u < 2), load_ahead=True,
                      gather_ahead=True)
            return carry

        lax.fori_loop(0, _STEADY, _six, 0)
        for j in range(6 * _STEADY, _NB):
            _slot(j, j % 6, guard_first=False, load_ahead=False,
                  gather_ahead=(j + 2 < _NB))
        _scatter_wait((_NB - 2) % 6, 0)
        _scatter_wait((_NB - 1) % 6, 1)
        plsc.subcore_barrier()

        for q in range(-(-nchunks // _TILES)):
            c = q * _TILES + tid

            @pl.when(c < nchunks)
            def _():
                pltpu.sync_copy(acc.at[pl.ds(c * _BW, _BW)], obuf.at[0, pl.ds(0, _BW)])
                pltpu.sync_copy(obuf.at[0, pl.ds(0, _BW)], out_h.at[pl.ds(cid * _N + c * _BW, _BW)])

    return k(hn2, idx4)


def _norm_relu(h, g, bb):
    mean = jnp.mean(h, axis=0, keepdims=True)
    var = jnp.mean((h - mean) ** 2, axis=0, keepdims=True)
    hn = (h - mean) * lax.rsqrt(var + 1e-5) * g + bb
    return jnp.maximum(hn, 0.0)


def _enc_body(x_ref, w_ref, b_ref, g_ref, bb_ref, h_ref, hn2_ref):
    h = jnp.dot(x_ref[...], w_ref[...], preferred_element_type=jnp.float32) + b_ref[...]
    h_ref[...] = h
    hn = _norm_relu(h, g_ref[...], bb_ref[...])
    hn2_ref[pl.ds(0, _N), :] = hn[:, :_HALF]
    hn2_ref[pl.ds(_N, _N), :] = hn[:, _HALF:]


def _agg_feats(s_ref, hn2_ref):
    s = s_ref[...]
    hn2 = hn2_ref[...]
    f0 = hn2[:_N] + s[:_N, :_HALF] / (s[:_N, _HALF:] + 1e-16)
    f1 = hn2[_N:] + s[_N:, :_HALF] / (s[_N:, _HALF:] + 1e-16)
    return f0, f1


def _mid_body(s_ref, hn2_ref, h_ref, w_ref, b_ref, g_ref, bb_ref, h_out, hn2_out):
    f0, f1 = _agg_feats(s_ref, hn2_ref)
    conv = (
        jnp.dot(f0, w_ref[: _HALF, :], preferred_element_type=jnp.float32)
        + jnp.dot(f1, w_ref[_HALF:, :], preferred_element_type=jnp.float32)
        + b_ref[...]
    )
    h_new = h_ref[...] + conv
    h_out[...] = h_new
    hn_new = _norm_relu(h_new, g_ref[...], bb_ref[...])
    hn2_out[pl.ds(0, _N), :] = hn_new[:, :_HALF]
    hn2_out[pl.ds(_N, _N), :] = hn_new[:, _HALF:]


def _last_body(s_ref, hn2_ref, h_ref, w_ref, b_ref, wo_ref, bo_ref, out_ref):
    f0, f1 = _agg_feats(s_ref, hn2_ref)
    conv = (
        jnp.dot(f0, w_ref[: _HALF, :], preferred_element_type=jnp.float32)
        + jnp.dot(f1, w_ref[_HALF:, :], preferred_element_type=jnp.float32)
        + b_ref[...]
    )
    h_new = h_ref[...] + conv
    logits = jnp.dot(h_new, wo_ref[...], preferred_element_type=jnp.float32) + bo_ref[...]
    mx = jnp.max(logits, axis=-1, keepdims=True)
    sh = logits - mx
    lse = jnp.log(jnp.sum(jnp.exp(sh), axis=-1, keepdims=True))
    out_ref[...] = sh - lse


def kernel(node_feats, edge_index, W_enc, b_enc, gamma, beta_bn, W_conv, b_conv, W_out, b_out):
    src = edge_index[0]
    dst = edge_index[1]
    src3 = jnp.stack([src, src + _N]).reshape(2, _TILES, _NB, _B)
    dst3 = jnp.broadcast_to(dst.reshape(1, _TILES, _NB, _B), (2, _TILES, _NB, _B))
    idx4 = jnp.stack([src3, dst3], axis=3)
    b_enc2 = b_enc.reshape(1, _HID)
    gamma2 = gamma.reshape(1, _HID)
    beta2 = beta_bn.reshape(1, _HID)

    h, hn2 = pl.pallas_call(
        _enc_body,
        out_shape=(
            jax.ShapeDtypeStruct((_N, _HID), jnp.float32),
            jax.ShapeDtypeStruct((2 * _N, _HALF), jnp.float32),
        ),
    )(node_feats, W_enc, b_enc2, gamma2, beta2)

    for i in range(3):
        s = _sc_edge_softmax(hn2, idx4)
        h, hn2 = pl.pallas_call(
            _mid_body,
            out_shape=(
                jax.ShapeDtypeStruct((_N, _HID), jnp.float32),
                jax.ShapeDtypeStruct((2 * _N, _HALF), jnp.float32),
            ),
        )(s, hn2, h, W_conv[i], b_conv[i].reshape(1, _HID), gamma2, beta2)

    s = _sc_edge_softmax(hn2, idx4)
    return pl.pallas_call(
        _last_body,
        out_shape=jax.ShapeDtypeStruct((_N, _OUT), jnp.float32),
    )(s, hn2, h, W_conv[3], b_conv[3].reshape(1, _HID), W_out, b_out.reshape(1, _OUT))


# speedup vs baseline: 1.1890x; 1.1322x over previous
import functools

import jax
import jax.numpy as jnp
from jax import lax
from jax.experimental import pallas as pl
from jax.experimental.pallas import tpu as pltpu
from jax.experimental.pallas import tpu_sc as plsc

_N = 10000
_E = 320000
_HID = 128
_HALF = 64
_OUT = 40
_TILES = 16
_B = 80
_EPT = _E // _TILES
_NB = _EPT // _B
_BW = 80
_EPS_GEN = 1e-7
_LANES = 16
_PI64 = tuple(
    32 * q + 2 * l + r for q in (0, 1) for r in (0, 1) for l in range(16)
)


def _sc_edge_softmax(hn2, idx4):
    mesh = plsc.VectorSubcoreMesh(core_axis_name="c", subcore_axis_name="s")
    assert _NB % 6 == 4 and _NB % 2 == 0
    _STEADY = (_NB - 4) // 6

    @functools.partial(
        pl.kernel,
        out_type=jax.ShapeDtypeStruct((2 * _N, _HID), jnp.float32),
        mesh=mesh,
        compiler_params=pltpu.CompilerParams(
            use_tc_tiling_on_sc=False, needs_layout_passes=False
        ),
        scratch_types=[
            pltpu.VMEM((6, 2, _B), jnp.int32),
            pltpu.VMEM((3, _B, _HALF), jnp.bfloat16),
            pltpu.VMEM((2, _B, _HID), jnp.float32),
            pltpu.VMEM_SHARED((_N, _HID), jnp.float32),
        ] + [pltpu.SemaphoreType.DMA] * 11,
    )
    def k(hn2_h, idx4_h, out_h, ring, grows, obuf, acc,
          is0, is1, is2, is3, is4, is5, gs0, gs1, gs2, ss0, ss1):
        cid = lax.axis_index("c")
        tid = lax.axis_index("s")
        isems = (is0, is1, is2, is3, is4, is5)
        gsems = (gs0, gs1, gs2)
        ssems = (ss0, ss1)

        def _zrow(i, carry):
            for q in range(_HID // _LANES):
                obuf[0, i, pl.ds(q * _LANES, _LANES)] = jnp.zeros((_LANES,), jnp.float32)
            return carry

        lax.fori_loop(0, _BW, _zrow, 0)
        nchunks = _N // _BW
        for q in range(-(-nchunks // _TILES)):
            c = q * _TILES + tid

            @pl.when(c < nchunks)
            def _():
                pltpu.sync_copy(obuf.at[0, pl.ds(0, _BW)], acc.at[pl.ds(c * _BW, _BW)])

        plsc.subcore_barrier()

        def _ring_load(j, slot):
            pltpu.async_copy(idx4_h.at[cid, tid, j], ring.at[slot], isems[slot])

        def _ring_wait(j, slot):
            pltpu.make_async_copy(idx4_h.at[cid, tid, j], ring.at[slot], isems[slot]).wait()

        def _gather(slot, gb):
            pltpu.async_copy(hn2_h.at[ring.at[slot, 0]], grows.at[gb], gsems[gb])

        def _gather_wait(slot, gb):
            pltpu.make_async_copy(hn2_h.at[ring.at[slot, 0]], grows.at[gb], gsems[gb]).wait()

        def _scatter(slot, b):
            pltpu.async_copy(obuf.at[b], acc.at[ring.at[slot, 1]], ssems[b], add=True)

        def _scatter_wait(slot, b):
            pltpu.make_async_copy(obuf.at[b], acc.at[ring.at[slot, 1]], ssems[b]).wait()

        def _slot(j, u, *, guard_first, load_ahead, gather_ahead):
            gb = u % 3
            b = u % 2
            _gather_wait(u, gb)
            if guard_first:
                @pl.when(j >= 2)
                def _():
                    _scatter_wait((u - 2) % 6, b)
            else:
                _scatter_wait((u - 2) % 6, b)
            if load_ahead:
                _ring_load(j + 4, (u + 4) % 6)
            if gather_ahead:
                _ring_wait(j + 2, (u + 2) % 6)
                _gather((u + 2) % 6, (u + 2) % 3)

            @plsc.parallel_loop(0, _B, unroll=2)
            def _row(i):
                for q in range(2):
                    raw = grows[gb, i, pl.ds(q * 32, 32)]
                    vi = plsc.bitcast(raw, jnp.int32)
                    fe = plsc.bitcast(vi << 16, jnp.float32)
                    fo = plsc.bitcast(vi & jnp.int32(-65536), jnp.float32)
                    me = fe + _EPS_GEN
                    mo = fo + _EPS_GEN
                    pe = jnp.exp(me)
                    po = jnp.exp(mo)
                    obuf[b, i, pl.ds(q * 32, _LANES)] = me * pe
                    obuf[b, i, pl.ds(q * 32 + 16, _LANES)] = mo * po
                    obuf[b, i, pl.ds(_HALF + q * 32, _LANES)] = pe
                    obuf[b, i, pl.ds(_HALF + q * 32 + 16, _LANES)] = po

            _scatter(u, b)

        for t in range(4):
            _ring_load(t, t)
        for t in range(2):
            _ring_wait(t, t)
            _gather(t, t)

        def _six(kk, carry):
            for u in range(6):
                _slot(6 * kk + u, u, guard_first=(u < 2), load_ahead=True,
                      gather_ahead=True)
            return carry

        lax.fori_loop(0, _STEADY, _six, 0)
        for j in range(6 * _STEADY, _NB):
            _slot(j, j % 6, guard_first=False, load_ahead=False,
                  gather_ahead=(j + 2 < _NB))
        _scatter_wait((_NB - 2) % 6, 0)
        _scatter_wait((_NB - 1) % 6, 1)
        plsc.subcore_barrier()

        for q in range(-(-nchunks // _TILES)):
            c = q * _TILES + tid

            @pl.when(c < nchunks)
            def _():
                pltpu.sync_copy(acc.at[pl.ds(c * _BW, _BW)], obuf.at[0, pl.ds(0, _BW)])
                pltpu.sync_copy(obuf.at[0, pl.ds(0, _BW)], out_h.at[pl.ds(cid * _N + c * _BW, _BW)])

    return k(hn2, idx4)


def _norm_relu(h, g, bb):
    mean = jnp.mean(h, axis=0, keepdims=True)
    var = jnp.mean((h - mean) ** 2, axis=0, keepdims=True)
    hn = (h - mean) * lax.rsqrt(var + 1e-5) * g + bb
    return jnp.maximum(hn, 0.0)


def _store_hn2(hn, hn2_ref):
    hn2_ref[pl.ds(0, _N), :] = hn[:, :_HALF].astype(jnp.bfloat16)
    hn2_ref[pl.ds(_N, _N), :] = hn[:, _HALF:].astype(jnp.bfloat16)


def _enc_body(x_ref, w_ref, b_ref, g_ref, bb_ref, h_ref, hn2_ref):
    h = jnp.dot(x_ref[...], w_ref[...], preferred_element_type=jnp.float32) + b_ref[...]
    h_ref[...] = h
    hn = _norm_relu(h, g_ref[...], bb_ref[...])
    _store_hn2(hn, hn2_ref)


def _conv_out(s_ref, hn, w_ref, w0_ref, w1_ref, b_ref):
    s = s_ref[...]
    agg0 = s[:_N, :_HALF] / (s[:_N, _HALF:] + 1e-16)
    agg1 = s[_N:, :_HALF] / (s[_N:, _HALF:] + 1e-16)
    return (
        jnp.dot(hn, w_ref[...], preferred_element_type=jnp.float32)
        + jnp.dot(agg0, w0_ref[...], preferred_element_type=jnp.float32)
        + jnp.dot(agg1, w1_ref[...], preferred_element_type=jnp.float32)
        + b_ref[...]
    )


def _mid_body(s_ref, h_ref, w_ref, w0_ref, w1_ref, b_ref, g_ref, bb_ref, h_out, hn2_out):
    h = h_ref[...]
    hn = _norm_relu(h, g_ref[...], bb_ref[...])
    h_new = h + _conv_out(s_ref, hn, w_ref, w0_ref, w1_ref, b_ref)
    h_out[...] = h_new
    hn_new = _norm_relu(h_new, g_ref[...], bb_ref[...])
    _store_hn2(hn_new, hn2_out)


def _last_body(s_ref, h_ref, w_ref, w0_ref, w1_ref, b_ref, g_ref, bb_ref, wo_ref, bo_ref, out_ref):
    h = h_ref[...]
    hn = _norm_relu(h, g_ref[...], bb_ref[...])
    h_new = h + _conv_out(s_ref, hn, w_ref, w0_ref, w1_ref, b_ref)
    logits = jnp.dot(h_new, wo_ref[...], preferred_element_type=jnp.float32) + bo_ref[...]
    mx = jnp.max(logits, axis=-1, keepdims=True)
    sh = logits - mx
    lse = jnp.log(jnp.sum(jnp.exp(sh), axis=-1, keepdims=True))
    out_ref[...] = sh - lse


def kernel(node_feats, edge_index, W_enc, b_enc, gamma, beta_bn, W_conv, b_conv, W_out, b_out):
    src = edge_index[0]
    dst = edge_index[1]
    src3 = jnp.stack([src, src + _N]).reshape(2, _TILES, _NB, _B)
    dst3 = jnp.broadcast_to(dst.reshape(1, _TILES, _NB, _B), (2, _TILES, _NB, _B))
    idx4 = jnp.stack([src3, dst3], axis=3)
    b_enc2 = b_enc.reshape(1, _HID)
    gamma2 = gamma.reshape(1, _HID)
    beta2 = beta_bn.reshape(1, _HID)
    pi = jnp.array(_PI64, jnp.int32)
    w0 = W_conv[:, pi, :]
    w1 = W_conv[:, pi + _HALF, :]

    h, hn2 = pl.pallas_call(
        _enc_body,
        out_shape=(
            jax.ShapeDtypeStruct((_N, _HID), jnp.float32),
            jax.ShapeDtypeStruct((2 * _N, _HALF), jnp.bfloat16),
        ),
    )(node_feats, W_enc, b_enc2, gamma2, beta2)

    for i in range(3):
        s = _sc_edge_softmax(hn2, idx4)
        h, hn2 = pl.pallas_call(
            _mid_body,
            out_shape=(
                jax.ShapeDtypeStruct((_N, _HID), jnp.float32),
                jax.ShapeDtypeStruct((2 * _N, _HALF), jnp.bfloat16),
            ),
        )(s, h, W_conv[i], w0[i], w1[i], b_conv[i].reshape(1, _HID), gamma2, beta2)

    s = _sc_edge_softmax(hn2, idx4)
    return pl.pallas_call(
        _last_body,
        out_shape=jax.ShapeDtypeStruct((_N, _OUT), jnp.float32),
    )(s, h, W_conv[3], w0[3], w1[3], b_conv[3].reshape(1, _HID), gamma2, beta2,
      W_out, b_out.reshape(1, _OUT))


# speedup vs baseline: 1.2248x; 1.0302x over previous
import functools

import jax
import jax.numpy as jnp
from jax import lax
from jax.experimental import pallas as pl
from jax.experimental.pallas import tpu as pltpu
from jax.experimental.pallas import tpu_sc as plsc

_N = 10000
_E = 320000
_HID = 128
_HALF = 64
_OUT = 40
_TILES = 16
_B = 80
_EPT = _E // _TILES
_NB = _EPT // _B
_BW = 80
_EPS_GEN = 1e-7
_LANES = 16
_PI64 = tuple(
    32 * q + 2 * l + r for q in (0, 1) for r in (0, 1) for l in range(16)
)


def _sc_edge_softmax(hn2, idx4):
    mesh = plsc.VectorSubcoreMesh(core_axis_name="c", subcore_axis_name="s")
    assert _NB % 6 == 4 and _NB % 2 == 0
    _STEADY = (_NB - 4) // 6

    @functools.partial(
        pl.kernel,
        out_type=jax.ShapeDtypeStruct((2 * _N, _HID), jnp.float32),
        mesh=mesh,
        compiler_params=pltpu.CompilerParams(
            use_tc_tiling_on_sc=False, needs_layout_passes=False
        ),
        scratch_types=[
            pltpu.VMEM((6, 2, _B), jnp.int32),
            pltpu.VMEM((3, _B, _HALF), jnp.bfloat16),
            pltpu.VMEM((3, _B, _HID), jnp.float32),
            pltpu.VMEM_SHARED((_N, _HID), jnp.float32),
        ] + [pltpu.SemaphoreType.DMA] * 12,
    )
    def k(hn2_h, idx4_h, out_h, ring, grows, obuf, acc,
          is0, is1, is2, is3, is4, is5, gs0, gs1, gs2, ss0, ss1, ss2):
        cid = lax.axis_index("c")
        tid = lax.axis_index("s")
        isems = (is0, is1, is2, is3, is4, is5)
        gsems = (gs0, gs1, gs2)
        ssems = (ss0, ss1, ss2)

        def _zrow(i, carry):
            for q in range(_HID // _LANES):
                obuf[0, i, pl.ds(q * _LANES, _LANES)] = jnp.zeros((_LANES,), jnp.float32)
            return carry

        lax.fori_loop(0, _BW, _zrow, 0)
        nchunks = _N // _BW
        for q in range(-(-nchunks // _TILES)):
            c = q * _TILES + tid

            @pl.when(c < nchunks)
            def _():
                pltpu.sync_copy(obuf.at[0, pl.ds(0, _BW)], acc.at[pl.ds(c * _BW, _BW)])

        plsc.subcore_barrier()

        def _ring_load(j, slot):
            pltpu.async_copy(idx4_h.at[cid, tid, j], ring.at[slot], isems[slot])

        def _ring_wait(j, slot):
            pltpu.make_async_copy(idx4_h.at[cid, tid, j], ring.at[slot], isems[slot]).wait()

        def _gather(slot, gb):
            pltpu.async_copy(hn2_h.at[ring.at[slot, 0]], grows.at[gb], gsems[gb])

        def _gather_wait(slot, gb):
            pltpu.make_async_copy(hn2_h.at[ring.at[slot, 0]], grows.at[gb], gsems[gb]).wait()

        def _scatter(slot, b):
            pltpu.async_copy(obuf.at[b], acc.at[ring.at[slot, 1]], ssems[b], add=True)

        def _scatter_wait(slot, b):
            pltpu.make_async_copy(obuf.at[b], acc.at[ring.at[slot, 1]], ssems[b]).wait()

        def _slot(j, u, *, guard_first, load_ahead, gather_ahead):
            gb = u % 3
            b = u % 3
            _gather_wait(u, gb)
            if guard_first:
                @pl.when(j >= 3)
                def _():
                    _scatter_wait((u - 3) % 6, b)
            else:
                _scatter_wait((u - 3) % 6, b)
            if load_ahead:
                _ring_load(j + 4, (u + 4) % 6)
            if gather_ahead:
                _ring_wait(j + 2, (u + 2) % 6)
                _gather((u + 2) % 6, (u + 2) % 3)

            @plsc.parallel_loop(0, _B, unroll=2)
            def _row(i):
                for q in range(2):
                    raw = grows[gb, i, pl.ds(q * 32, 32)]
                    vi = plsc.bitcast(raw, jnp.int32)
                    fe = plsc.bitcast(vi << 16, jnp.float32)
                    fo = plsc.bitcast(vi & jnp.int32(-65536), jnp.float32)
                    me = fe + _EPS_GEN
                    mo = fo + _EPS_GEN
                    pe = jnp.exp(me)
                    po = jnp.exp(mo)
                    obuf[b, i, pl.ds(q * 32, _LANES)] = me * pe
                    obuf[b, i, pl.ds(q * 32 + 16, _LANES)] = mo * po
                    obuf[b, i, pl.ds(_HALF + q * 32, _LANES)] = pe
                    obuf[b, i, pl.ds(_HALF + q * 32 + 16, _LANES)] = po

            _scatter(u, b)

        for t in range(4):
            _ring_load(t, t)
        for t in range(2):
            _ring_wait(t, t)
            _gather(t, t)

        def _six(kk, carry):
            for u in range(6):
                _slot(6 * kk + u, u, guard_first=(u < 3), load_ahead=True,
                      gather_ahead=True)
            return carry

        lax.fori_loop(0, _STEADY, _six, 0)
        for j in range(6 * _STEADY, _NB):
            _slot(j, j % 6, guard_first=False, load_ahead=False,
                  gather_ahead=(j + 2 < _NB))
        for j in range(_NB - 3, _NB):
            _scatter_wait(j % 6, j % 3)
        plsc.subcore_barrier()

        for q in range(-(-nchunks // _TILES)):
            c = q * _TILES + tid

            @pl.when(c < nchunks)
            def _():
                pltpu.sync_copy(acc.at[pl.ds(c * _BW, _BW)], obuf.at[0, pl.ds(0, _BW)])
                pltpu.sync_copy(obuf.at[0, pl.ds(0, _BW)], out_h.at[pl.ds(cid * _N + c * _BW, _BW)])

    return k(hn2, idx4)


def _norm_relu(h, g, bb):
    mean = jnp.mean(h, axis=0, keepdims=True)
    var = jnp.mean((h - mean) ** 2, axis=0, keepdims=True)
    hn = (h - mean) * lax.rsqrt(var + 1e-5) * g + bb
    return jnp.maximum(hn, 0.0)


def _store_hn2(hn, hn2_ref):
    hn2_ref[pl.ds(0, _N), :] = hn[:, :_HALF].astype(jnp.bfloat16)
    hn2_ref[pl.ds(_N, _N), :] = hn[:, _HALF:].astype(jnp.bfloat16)


def _enc_body(x_ref, w_ref, b_ref, g_ref, bb_ref, h_ref, hn2_ref):
    h = jnp.dot(x_ref[...], w_ref[...], preferred_element_type=jnp.float32) + b_ref[...]
    h_ref[...] = h
    hn = _norm_relu(h, g_ref[...], bb_ref[...])
    _store_hn2(hn, hn2_ref)


def _conv_out(s_ref, hn, w_ref, w0_ref, w1_ref, b_ref):
    s = s_ref[...]
    agg0 = s[:_N, :_HALF] / (s[:_N, _HALF:] + 1e-16)
    agg1 = s[_N:, :_HALF] / (s[_N:, _HALF:] + 1e-16)
    return (
        jnp.dot(hn, w_ref[...], preferred_element_type=jnp.float32)
        + jnp.dot(agg0, w0_ref[...], preferred_element_type=jnp.float32)
        + jnp.dot(agg1, w1_ref[...], preferred_element_type=jnp.float32)
        + b_ref[...]
    )


def _mid_body(s_ref, h_ref, w_ref, w0_ref, w1_ref, b_ref, g_ref, bb_ref, h_out, hn2_out):
    h = h_ref[...]
    hn = _norm_relu(h, g_ref[...], bb_ref[...])
    h_new = h + _conv_out(s_ref, hn, w_ref, w0_ref, w1_ref, b_ref)
    h_out[...] = h_new
    hn_new = _norm_relu(h_new, g_ref[...], bb_ref[...])
    _store_hn2(hn_new, hn2_out)


def _last_body(s_ref, h_ref, w_ref, w0_ref, w1_ref, b_ref, g_ref, bb_ref, wo_ref, bo_ref, out_ref):
    h = h_ref[...]
    hn = _norm_relu(h, g_ref[...], bb_ref[...])
    h_new = h + _conv_out(s_ref, hn, w_ref, w0_ref, w1_ref, b_ref)
    logits = jnp.dot(h_new, wo_ref[...], preferred_element_type=jnp.float32) + bo_ref[...]
    mx = jnp.max(logits, axis=-1, keepdims=True)
    sh = logits - mx
    lse = jnp.log(jnp.sum(jnp.exp(sh), axis=-1, keepdims=True))
    out_ref[...] = sh - lse


def kernel(node_feats, edge_index, W_enc, b_enc, gamma, beta_bn, W_conv, b_conv, W_out, b_out):
    src = edge_index[0]
    dst = edge_index[1]
    src3 = jnp.stack([src, src + _N]).reshape(2, _TILES, _NB, _B)
    dst3 = jnp.broadcast_to(dst.reshape(1, _TILES, _NB, _B), (2, _TILES, _NB, _B))
    idx4 = jnp.stack([src3, dst3], axis=3)
    b_enc2 = b_enc.reshape(1, _HID)
    gamma2 = gamma.reshape(1, _HID)
    beta2 = beta_bn.reshape(1, _HID)
    pi = jnp.array(_PI64, jnp.int32)
    w0 = W_conv[:, pi, :]
    w1 = W_conv[:, pi + _HALF, :]

    h, hn2 = pl.pallas_call(
        _enc_body,
        out_shape=(
            jax.ShapeDtypeStruct((_N, _HID), jnp.float32),
            jax.ShapeDtypeStruct((2 * _N, _HALF), jnp.bfloat16),
        ),
    )(node_feats, W_enc, b_enc2, gamma2, beta2)

    for i in range(3):
        s = _sc_edge_softmax(hn2, idx4)
        h, hn2 = pl.pallas_call(
            _mid_body,
            out_shape=(
                jax.ShapeDtypeStruct((_N, _HID), jnp.float32),
                jax.ShapeDtypeStruct((2 * _N, _HALF), jnp.bfloat16),
            ),
        )(s, h, W_conv[i], w0[i], w1[i], b_conv[i].reshape(1, _HID), gamma2, beta2)

    s = _sc_edge_softmax(hn2, idx4)
    return pl.pallas_call(
        _last_body,
        out_shape=jax.ShapeDtypeStruct((_N, _OUT), jnp.float32),
    )(s, h, W_conv[3], w0[3], w1[3], b_conv[3].reshape(1, _HID), gamma2, beta2,
      W_out, b_out.reshape(1, _OUT))


# speedup vs baseline: 1.2334x; 1.0070x over previous
import functools

import jax
import jax.numpy as jnp
from jax import lax
from jax.experimental import pallas as pl
from jax.experimental.pallas import tpu as pltpu
from jax.experimental.pallas import tpu_sc as plsc

_N = 10000
_E = 320000
_HID = 128
_HALF = 64
_OUT = 40
_TILES = 16
_B = 80
_EPT = _E // _TILES
_NB = _EPT // _B
_BW = 80
_EPS_GEN = 1e-7
_LANES = 16
_PI64 = tuple(
    32 * q + 2 * l + r for q in (0, 1) for r in (0, 1) for l in range(16)
)


def _sc_edge_softmax(hn2, idx4):
    mesh = plsc.VectorSubcoreMesh(core_axis_name="c", subcore_axis_name="s")
    assert _NB % 6 == 4 and _NB % 2 == 0
    _STEADY = (_NB - 4) // 6

    @functools.partial(
        pl.kernel,
        out_type=jax.ShapeDtypeStruct((2 * _N, _HID), jnp.float32),
        mesh=mesh,
        compiler_params=pltpu.CompilerParams(
            use_tc_tiling_on_sc=False, needs_layout_passes=False
        ),
        scratch_types=[
            pltpu.VMEM((6, 2, _B), jnp.int32),
            pltpu.VMEM((3, _B, _HALF), jnp.bfloat16),
            pltpu.VMEM((3, _B, _HID), jnp.float32),
            pltpu.VMEM_SHARED((_N, _HID), jnp.float32),
        ] + [pltpu.SemaphoreType.DMA] * 12,
    )
    def k(hn2_h, idx4_h, out_h, ring, grows, obuf, acc,
          is0, is1, is2, is3, is4, is5, gs0, gs1, gs2, ss0, ss1, ss2):
        cid = lax.axis_index("c")
        tid = lax.axis_index("s")
        isems = (is0, is1, is2, is3, is4, is5)
        gsems = (gs0, gs1, gs2)
        ssems = (ss0, ss1, ss2)

        def _zrow(i, carry):
            for q in range(_HID // _LANES):
                obuf[0, i, pl.ds(q * _LANES, _LANES)] = jnp.zeros((_LANES,), jnp.float32)
            return carry

        lax.fori_loop(0, _BW, _zrow, 0)
        nchunks = _N // _BW
        for q in range(-(-nchunks // _TILES)):
            c = q * _TILES + tid

            @pl.when(c < nchunks)
            def _():
                pltpu.sync_copy(obuf.at[0, pl.ds(0, _BW)], acc.at[pl.ds(c * _BW, _BW)])

        plsc.subcore_barrier()

        def _ring_load(j, slot):
            pltpu.async_copy(idx4_h.at[cid, tid, j], ring.at[slot], isems[slot])

        def _ring_wait(j, slot):
            pltpu.make_async_copy(idx4_h.at[cid, tid, j], ring.at[slot], isems[slot]).wait()

        def _gather(slot, gb):
            pltpu.async_copy(hn2_h.at[ring.at[slot, 0]], grows.at[gb], gsems[gb])

        def _gather_wait(slot, gb):
            pltpu.make_async_copy(hn2_h.at[ring.at[slot, 0]], grows.at[gb], gsems[gb]).wait()

        def _scatter(slot, b):
            pltpu.async_copy(obuf.at[b], acc.at[ring.at[slot, 1]], ssems[b], add=True)

        def _scatter_wait(slot, b):
            pltpu.make_async_copy(obuf.at[b], acc.at[ring.at[slot, 1]], ssems[b]).wait()

        def _slot(j, u, *, guard_first, load_ahead, gather_ahead):
            gb = u % 3
            b = u % 3
            _gather_wait(u, gb)
            if guard_first:
                @pl.when(j >= 3)
                def _():
                    _scatter_wait((u - 3) % 6, b)
            else:
                _scatter_wait((u - 3) % 6, b)
            if load_ahead:
                _ring_load(j + 4, (u + 4) % 6)
            if gather_ahead:
                _ring_wait(j + 2, (u + 2) % 6)
                _gather((u + 2) % 6, (u + 2) % 3)

            @plsc.parallel_loop(0, _B, unroll=2)
            def _row(i):
                for q in range(2):
                    raw = grows[gb, i, pl.ds(q * 32, 32)]
                    vi = plsc.bitcast(raw, jnp.int32)
                    fe = plsc.bitcast(vi << 16, jnp.float32)
                    fo = plsc.bitcast(vi & jnp.int32(-65536), jnp.float32)
                    me = fe + _EPS_GEN
                    mo = fo + _EPS_GEN
                    pe = jnp.exp(me)
                    po = jnp.exp(mo)
                    obuf[b, i, pl.ds(q * 32, _LANES)] = me * pe
                    obuf[b, i, pl.ds(q * 32 + 16, _LANES)] = mo * po
                    obuf[b, i, pl.ds(_HALF + q * 32, _LANES)] = pe
                    obuf[b, i, pl.ds(_HALF + q * 32 + 16, _LANES)] = po

            _scatter(u, b)

        for t in range(4):
            _ring_load(t, t)
        for t in range(2):
            _ring_wait(t, t)
            _gather(t, t)

        def _six(kk, carry):
            for u in range(6):
                _slot(6 * kk + u, u, guard_first=(u < 3), load_ahead=True,
                      gather_ahead=True)
            return carry

        lax.fori_loop(0, _STEADY, _six, 0)
        for j in range(6 * _STEADY, _NB):
            _slot(j, j % 6, guard_first=False, load_ahead=False,
                  gather_ahead=(j + 2 < _NB))
        for j in range(_NB - 3, _NB):
            _scatter_wait(j % 6, j % 3)
        plsc.subcore_barrier()

        for q in range(-(-nchunks // _TILES)):
            c = q * _TILES + tid

            @pl.when(c < nchunks)
            def _():
                pltpu.sync_copy(acc.at[pl.ds(c * _BW, _BW)], out_h.at[pl.ds(cid * _N + c * _BW, _BW)])

    return k(hn2, idx4)


def _norm_relu(h, g, bb):
    mean = jnp.mean(h, axis=0, keepdims=True)
    var = jnp.mean((h - mean) ** 2, axis=0, keepdims=True)
    hn = (h - mean) * lax.rsqrt(var + 1e-5) * g + bb
    return jnp.maximum(hn, 0.0)


def _store_hn2(hn, hn2_ref):
    hn2_ref[pl.ds(0, _N), :] = hn[:, :_HALF].astype(jnp.bfloat16)
    hn2_ref[pl.ds(_N, _N), :] = hn[:, _HALF:].astype(jnp.bfloat16)


def _enc_body(x_ref, w_ref, b_ref, g_ref, bb_ref, h_ref, hn2_ref):
    h = jnp.dot(x_ref[...], w_ref[...], preferred_element_type=jnp.float32) + b_ref[...]
    h_ref[...] = h
    hn = _norm_relu(h, g_ref[...], bb_ref[...])
    _store_hn2(hn, hn2_ref)


def _conv_out(s_ref, hn, w_ref, w0_ref, w1_ref, b_ref):
    s = s_ref[...]
    agg0 = s[:_N, :_HALF] / (s[:_N, _HALF:] + 1e-16)
    agg1 = s[_N:, :_HALF] / (s[_N:, _HALF:] + 1e-16)
    return (
        jnp.dot(hn, w_ref[...], preferred_element_type=jnp.float32)
        + jnp.dot(agg0, w0_ref[...], preferred_element_type=jnp.float32)
        + jnp.dot(agg1, w1_ref[...], preferred_element_type=jnp.float32)
        + b_ref[...]
    )


def _mid_body(s_ref, h_ref, w_ref, w0_ref, w1_ref, b_ref, g_ref, bb_ref, h_out, hn2_out):
    h = h_ref[...]
    hn = _norm_relu(h, g_ref[...], bb_ref[...])
    h_new = h + _conv_out(s_ref, hn, w_ref, w0_ref, w1_ref, b_ref)
    h_out[...] = h_new
    hn_new = _norm_relu(h_new, g_ref[...], bb_ref[...])
    _store_hn2(hn_new, hn2_out)


def _last_body(s_ref, h_ref, w_ref, w0_ref, w1_ref, b_ref, g_ref, bb_ref, wo_ref, bo_ref, out_ref):
    h = h_ref[...]
    hn = _norm_relu(h, g_ref[...], bb_ref[...])
    h_new = h + _conv_out(s_ref, hn, w_ref, w0_ref, w1_ref, b_ref)
    logits = jnp.dot(h_new, wo_ref[...], preferred_element_type=jnp.float32) + bo_ref[...]
    mx = jnp.max(logits, axis=-1, keepdims=True)
    sh = logits - mx
    lse = jnp.log(jnp.sum(jnp.exp(sh), axis=-1, keepdims=True))
    out_ref[...] = sh - lse


def kernel(node_feats, edge_index, W_enc, b_enc, gamma, beta_bn, W_conv, b_conv, W_out, b_out):
    src = edge_index[0]
    dst = edge_index[1]
    src3 = jnp.stack([src, src + _N]).reshape(2, _TILES, _NB, _B)
    dst3 = jnp.broadcast_to(dst.reshape(1, _TILES, _NB, _B), (2, _TILES, _NB, _B))
    idx4 = jnp.stack([src3, dst3], axis=3)
    b_enc2 = b_enc.reshape(1, _HID)
    gamma2 = gamma.reshape(1, _HID)
    beta2 = beta_bn.reshape(1, _HID)
    pi = jnp.array(_PI64, jnp.int32)
    w0 = W_conv[:, pi, :]
    w1 = W_conv[:, pi + _HALF, :]

    h, hn2 = pl.pallas_call(
        _enc_body,
        out_shape=(
            jax.ShapeDtypeStruct((_N, _HID), jnp.float32),
            jax.ShapeDtypeStruct((2 * _N, _HALF), jnp.bfloat16),
        ),
    )(node_feats, W_enc, b_enc2, gamma2, beta2)

    for i in range(3):
        s = _sc_edge_softmax(hn2, idx4)
        h, hn2 = pl.pallas_call(
            _mid_body,
            out_shape=(
                jax.ShapeDtypeStruct((_N, _HID), jnp.float32),
                jax.ShapeDtypeStruct((2 * _N, _HALF), jnp.bfloat16),
            ),
        )(s, h, W_conv[i], w0[i], w1[i], b_conv[i].reshape(1, _HID), gamma2, beta2)

    s = _sc_edge_softmax(hn2, idx4)
    return pl.pallas_call(
        _last_body,
        out_shape=jax.ShapeDtypeStruct((_N, _OUT), jnp.float32),
    )(s, h, W_conv[3], w0[3], w1[3], b_conv[3].reshape(1, _HID), gamma2, beta2,
      W_out, b_out.reshape(1, _OUT))
